# baseline (speedup 1.0000x reference)
import functools

import jax
import jax.numpy as jnp
from jax import lax
from jax.experimental import pallas as pl
from jax.experimental.pallas import tpu as pltpu
from jax.experimental.pallas import tpu_sc as plsc

B = 1024
BAG = 50
D = 4096
NE = 4096
C = 4
DC = D // C
NC = 2
NS = 16
NW = NC * NS
BW = B // NW

_MESH = plsc.VectorSubcoreMesh(core_axis_name="c", subcore_axis_name="s")


def _sc_body(table2, idxg, idxo, out2, idx_l, oidx_l, gbuf, acc, sem):
    w = lax.axis_index("s") * NC + lax.axis_index("c")
    pltpu.sync_copy(idxg.at[w], idx_l)
    pltpu.sync_copy(idxo.at[w], oidx_l)
    for c in range(C):
        pltpu.async_copy(table2.at[idx_l.at[c, 0]], acc, sem).wait()

        def step(l, carry, c=c):
            pltpu.async_copy(table2.at[idx_l.at[c, l]], gbuf, sem).wait()

            def row(r, carry2):
                @plsc.parallel_loop(0, DC, step=16)
                def _(k):
                    plsc.addupdate(acc.at[r, pl.ds(k, 16)],
                                   gbuf[r, pl.ds(k, 16)])

                return carry2

            lax.fori_loop(0, BW, row, 0)
            return carry

        lax.fori_loop(1, BAG, step, 0)
        pltpu.async_copy(acc, out2.at[oidx_l.at[c]], sem).wait()


_sc_call = pl.kernel(
    _sc_body,
    out_type=jax.ShapeDtypeStruct((B * C, DC), jnp.float32),
    mesh=_MESH,
    scratch_types=[
        pltpu.VMEM((C, BAG, BW), jnp.int32),
        pltpu.VMEM((C, BW), jnp.int32),
        pltpu.VMEM((BW, DC), jnp.float32),
        pltpu.VMEM((BW, DC), jnp.float32),
        pltpu.SemaphoreType.DMA,
    ],
)


@jax.jit
def kernel(codes, table):
    codes = codes.astype(jnp.int32)
    table2 = table.reshape(NE * C, DC)
    cp = codes.reshape(NW, BW, BAG).transpose(0, 2, 1)
    cvec = jnp.arange(C, dtype=jnp.int32)
    idxg = cp[:, None, :, :] * C + cvec[None, :, None, None]
    bag = jnp.arange(B, dtype=jnp.int32).reshape(NW, 1, BW)
    idxo = C * bag + cvec[None, :, None]
    out2 = _sc_call(table2, idxg, idxo)
    return out2.reshape(B, D)
